# Initial kernel scaffold; baseline (speedup 1.0000x reference)
import jax
import jax.numpy as jnp
from jax.experimental import pallas as pl


def kernel(src, tgt, src_embedding, tgt_embedding, positive_T):
    raise NotImplementedError("write your pallas kernel here")


# speedup vs baseline: 1.4935x; 1.4935x over previous
import jax
import jax.numpy as jnp
from jax.experimental import pallas as pl
from jax.experimental.pallas import tpu as pltpu

_N = 2048
_CE = 64
_BLK = 512
_EPS = 1e-05


def _corr_body(qe_ref, ke_ref, v_ref, corr_ref):
    q = qe_ref[0]
    k = ke_ref[0]
    v = v_ref[0]
    yy = jnp.sum(k * k, axis=0, keepdims=True)
    logits = 2.0 * jax.lax.dot_general(
        q, k, (((0,), (0,)), ((), ())),
        preferred_element_type=jnp.float32)
    logits = logits - yy
    m = jnp.max(logits, axis=1, keepdims=True)
    p = jnp.exp(logits - m)
    ones = jnp.ones((1, p.shape[1]), dtype=p.dtype)
    v4 = jnp.concatenate([v, ones], axis=0)
    acc = jax.lax.dot_general(
        v4, p, (((1,), (1,)), ((), ())),
        preferred_element_type=jnp.float32)
    corr_ref[0] = acc[:3, :] / acc[3:4, :]


def _cov_body(a_ref, b_ref, cov_ref, ca_ref, cb_ref):
    n = a_ref.shape[-1]
    w = (1.0 / n) / (1.0 + _EPS)
    a = a_ref[0]
    b = b_ref[0]
    ca = jnp.sum(a, axis=1, keepdims=True) * w
    cb = jnp.sum(b, axis=1, keepdims=True) * w
    ac = a - ca
    bc = b - cb
    cov = jax.lax.dot_general(
        ac, bc, (((1,), (1,)), ((), ())),
        preferred_element_type=jnp.float32) * w
    cov_ref[0] = cov
    ca_ref[0, 0] = ca[:, 0]
    cb_ref[0, 0] = cb[:, 0]


def kernel(src, tgt, src_embedding, tgt_embedding, positive_T):
    batch, posi_num, num_points, cdim = tgt.shape
    bp = batch * posi_num
    cemb = tgt_embedding.shape[2]

    src_ = jnp.swapaxes(src, -2, -1).reshape(bp, cdim, num_points)
    tgt_ = jnp.swapaxes(tgt, -2, -1).reshape(bp, cdim, num_points)
    src_emb = jnp.broadcast_to(
        jnp.squeeze(src_embedding, -1),
        (batch, posi_num, cemb, num_points)).reshape(bp, cemb, num_points)
    tgt_emb = jnp.squeeze(tgt_embedding, -1).reshape(bp, cemb, num_points)

    nblk = num_points // _BLK
    corr = pl.pallas_call(
        _corr_body,
        grid=(bp, nblk),
        in_specs=[
            pl.BlockSpec((1, cemb, _BLK), lambda b, j: (b, 0, j)),
            pl.BlockSpec((1, cemb, num_points), lambda b, j: (b, 0, 0)),
            pl.BlockSpec((1, cdim, num_points), lambda b, j: (b, 0, 0)),
        ],
        out_specs=pl.BlockSpec((1, cdim, _BLK), lambda b, j: (b, 0, j)),
        out_shape=jax.ShapeDtypeStruct((bp, cdim, num_points), jnp.float32),
        compiler_params=pltpu.CompilerParams(
            dimension_semantics=("parallel", "parallel")),
    )(src_emb, tgt_emb, tgt_)

    cov, ca, cb = pl.pallas_call(
        _cov_body,
        grid=(bp,),
        in_specs=[
            pl.BlockSpec((1, cdim, num_points), lambda b: (b, 0, 0)),
            pl.BlockSpec((1, cdim, num_points), lambda b: (b, 0, 0)),
        ],
        out_specs=[
            pl.BlockSpec((1, cdim, cdim), lambda b: (b, 0, 0)),
            pl.BlockSpec((1, 1, cdim), lambda b: (b, 0, 0)),
            pl.BlockSpec((1, 1, cdim), lambda b: (b, 0, 0)),
        ],
        out_shape=[
            jax.ShapeDtypeStruct((bp, cdim, cdim), jnp.float32),
            jax.ShapeDtypeStruct((bp, 1, cdim), jnp.float32),
            jax.ShapeDtypeStruct((bp, 1, cdim), jnp.float32),
        ],
        compiler_params=pltpu.CompilerParams(
            dimension_semantics=("parallel",)),
    )(src_, corr)

    u, _, vh = jnp.linalg.svd(cov, full_matrices=True)
    v = jnp.swapaxes(vh, -1, -2)
    rot_pos = v @ jnp.swapaxes(u, -1, -2)
    v_neg = v.at[:, :, 2].multiply(-1.0)
    rot_neg = v_neg @ jnp.swapaxes(u, -1, -2)
    det = jnp.linalg.det(rot_pos)
    rot_mat = jnp.where(det[:, None, None] > 0, rot_pos, rot_neg)
    ca = ca.reshape(bp, cdim)
    cb = cb.reshape(bp, cdim)
    translation = (-rot_mat @ ca[:, :, None] + cb[:, :, None]).reshape(bp, 3)
    return (rot_mat, translation, src_, corr)
